# baseline (speedup 1.0000x reference)
import functools

import jax
import jax.numpy as jnp
from jax import lax
from jax.experimental import pallas as pl
from jax.experimental.pallas import tpu as pltpu
from jax.experimental.pallas import tpu_sc as plsc

_BATCH = 16384
_D = 64
_NCLS = 100000
_L = 16

_info = plsc.get_sparse_core_info()
_NC, _NS = _info.num_cores, _info.num_subcores
_NW = _NC * _NS
_ROUNDS = _D // _NW
_LCH = 4096
_NLCH = _BATCH // _LCH
_STEPS = _ROUNDS * _NLCH


@functools.partial(
    pl.kernel,
    mesh=plsc.VectorSubcoreMesh(core_axis_name="c", subcore_axis_name="s"),
    out_type=jax.ShapeDtypeStruct((_NW, _L), jnp.float32),
    scratch_types=[
        pltpu.VMEM((_NCLS,), jnp.float32),
        pltpu.VMEM((_LCH,), jnp.float32),
        pltpu.VMEM((_LCH,), jnp.float32),
        pltpu.VMEM((_LCH,), jnp.int32),
        pltpu.VMEM((_LCH,), jnp.int32),
        pltpu.VMEM((_L,), jnp.float32),
        pltpu.SemaphoreType.DMA,
        pltpu.SemaphoreType.DMA,
        pltpu.SemaphoreType.DMA,
        pltpu.SemaphoreType.DMA,
        pltpu.SemaphoreType.DMA,
    ],
    compiler_params=pltpu.CompilerParams(
        use_tc_tiling_on_sc=True, needs_layout_passes=False),
)
def _center_loss_sc(features_hbm, labels_hbm, centers_hbm, out_hbm,
                    row_v, feat_a, feat_b, lab_a, lab_b, acc_v,
                    rsem, fsem_a, fsem_b, lsem_a, lsem_b):
    wid = lax.axis_index("s") * _NC + lax.axis_index("c")

    labs = (lab_a, lab_b)
    lsems = (lsem_a, lsem_b)
    feats = (feat_a, feat_b)
    fsems = (fsem_a, fsem_b)

    def start_step(step):
        r, k = divmod(step, _NLCH)
        d = wid * _ROUNDS + r
        return (
            pltpu.async_copy(labels_hbm.at[pl.ds(k * _LCH, _LCH)],
                             labs[step % 2], lsems[step % 2]),
            pltpu.async_copy(features_hbm.at[d, pl.ds(k * _LCH, _LCH)],
                             feats[step % 2], fsems[step % 2]),
        )

    copies = [start_step(0)]
    rcopy = pltpu.async_copy(centers_hbm.at[wid * _ROUNDS, :], row_v, rsem)

    zero = jnp.zeros((_L,), jnp.float32)
    accs = (zero,) * 4

    for step in range(_STEPS):
        r, k = divmod(step, _NLCH)
        buf, fbuf = labs[step % 2], feats[step % 2]
        lc, fc = copies[step]
        lc.wait()
        fc.wait()
        if step + 1 < _STEPS:
            copies.append(start_step(step + 1))
        if k == 0:
            rcopy.wait()

        def body(v, acc4, _buf=buf, _fbuf=fbuf):
            l16 = _buf[pl.ds(v * _L, _L)]
            c = plsc.load_gather(row_v, [l16])
            f = _fbuf[pl.ds(v * _L, _L)]
            d = f - c
            b0, b1, b2, b3 = acc4
            return (b1, b2, b3, b0 + d * d)

        accs = lax.fori_loop(0, _LCH // _L, body, accs, unroll=16)

        if k == _NLCH - 1 and r + 1 < _ROUNDS:
            rcopy = pltpu.async_copy(
                centers_hbm.at[wid * _ROUNDS + r + 1, :], row_v, rsem)

    acc_v[...] = (accs[0] + accs[1]) + (accs[2] + accs[3])
    pltpu.sync_copy(acc_v, out_hbm.at[wid])


def kernel(features, labels, centers):
    partials = _center_loss_sc(
        features.T,
        labels.astype(jnp.int32),
        centers.T,
    )
    return jnp.sum(partials) * (0.5 / _BATCH)


# speedup vs baseline: 1.0242x; 1.0242x over previous
import functools

import jax
import jax.numpy as jnp
from jax import lax
from jax.experimental import pallas as pl
from jax.experimental.pallas import tpu as pltpu
from jax.experimental.pallas import tpu_sc as plsc

_BATCH = 16384
_D = 64
_NCLS = 100000
_L = 16

_info = plsc.get_sparse_core_info()
_NC, _NS = _info.num_cores, _info.num_subcores
_NW = _NC * _NS
_ROUNDS = _D // _NW
_LCH = 4096
_NLCH = _BATCH // _LCH
_STEPS = _ROUNDS * _NLCH


@functools.partial(
    pl.kernel,
    mesh=plsc.VectorSubcoreMesh(core_axis_name="c", subcore_axis_name="s"),
    out_type=jax.ShapeDtypeStruct((_NW, _L), jnp.float32),
    scratch_types=[
        pltpu.VMEM((_NCLS,), jnp.float32),
        pltpu.VMEM((_LCH,), jnp.float32),
        pltpu.VMEM((_LCH,), jnp.float32),
        pltpu.VMEM((_LCH,), jnp.int32),
        pltpu.VMEM((_LCH,), jnp.int32),
        pltpu.VMEM((_L,), jnp.float32),
        pltpu.SemaphoreType.DMA,
        pltpu.SemaphoreType.DMA,
        pltpu.SemaphoreType.DMA,
        pltpu.SemaphoreType.DMA,
        pltpu.SemaphoreType.DMA,
    ],
    compiler_params=pltpu.CompilerParams(
        use_tc_tiling_on_sc=True, needs_layout_passes=False),
)
def _center_loss_sc(features_hbm, labels_hbm, centers_hbm, out_hbm,
                    row_v, feat_a, feat_b, lab_a, lab_b, acc_v,
                    rsem, fsem_a, fsem_b, lsem_a, lsem_b):
    wid = lax.axis_index("s") * _NC + lax.axis_index("c")

    labs = (lab_a, lab_b)
    lsems = (lsem_a, lsem_b)
    feats = (feat_a, feat_b)
    fsems = (fsem_a, fsem_b)

    def start_step(step):
        r, k = divmod(step, _NLCH)
        d = wid * _ROUNDS + r
        return (
            pltpu.async_copy(labels_hbm.at[pl.ds(k * _LCH, _LCH)],
                             labs[step % 2], lsems[step % 2]),
            pltpu.async_copy(features_hbm.at[d, pl.ds(k * _LCH, _LCH)],
                             feats[step % 2], fsems[step % 2]),
        )

    copies = [start_step(0)]
    rcopy = pltpu.async_copy(centers_hbm.at[wid * _ROUNDS, :], row_v, rsem)

    zero = jnp.zeros((_L,), jnp.float32)
    accs = (zero,) * 4

    for step in range(_STEPS):
        r, k = divmod(step, _NLCH)
        buf, fbuf = labs[step % 2], feats[step % 2]
        lc, fc = copies[step]
        lc.wait()
        fc.wait()
        if step + 1 < _STEPS:
            copies.append(start_step(step + 1))
        if k == 0:
            rcopy.wait()

        @plsc.parallel_loop(0, _LCH // _L, unroll=8, carry=accs)
        def accs(v, acc4, _buf=buf, _fbuf=fbuf):
            l16 = _buf[pl.ds(v * _L, _L)]
            c = plsc.load_gather(row_v, [l16])
            f = _fbuf[pl.ds(v * _L, _L)]
            d = f - c
            b0, b1, b2, b3 = acc4
            return (b1, b2, b3, b0 + d * d)

        if k == _NLCH - 1 and r + 1 < _ROUNDS:
            rcopy = pltpu.async_copy(
                centers_hbm.at[wid * _ROUNDS + r + 1, :], row_v, rsem)

    acc_v[...] = (accs[0] + accs[1]) + (accs[2] + accs[3])
    pltpu.sync_copy(acc_v, out_hbm.at[wid])


def kernel(features, labels, centers):
    partials = _center_loss_sc(
        features.T,
        labels.astype(jnp.int32),
        centers.T,
    )
    return jnp.sum(partials) * (0.5 / _BATCH)
